# baseline (speedup 1.0000x reference)
import functools
import math

import jax
import jax.numpy as jnp
from jax import lax
from jax.experimental import pallas as pl
from jax.experimental.pallas import tpu as pltpu
from jax.experimental.pallas import tpu_sc as plsc

D_MODEL = 128
SCALE = math.sqrt(D_MODEL)

_NC = 2
_NS = 16
_NW = _NC * _NS

_CH = 128
_NBUF = 5
_A = 2

_TC_BLOCK = 4000


def _scale_body(t_ref, o_ref):
    o_ref[...] = t_ref[...] * SCALE


@functools.lru_cache(maxsize=None)
def _make_scale(V: int):
    assert V % _TC_BLOCK == 0
    return pl.pallas_call(
        _scale_body,
        grid=(V // _TC_BLOCK,),
        in_specs=[pl.BlockSpec((_TC_BLOCK, D_MODEL), lambda i: (i, 0))],
        out_specs=pl.BlockSpec((_TC_BLOCK, D_MODEL), lambda i: (i, 0)),
        out_shape=jax.ShapeDtypeStruct((V, D_MODEL), jnp.float32),
    )


@functools.lru_cache(maxsize=None)
def _make_gather(B: int):
    assert B % (_NW * _CH * _NBUF) == 0
    n_per_w = B // _NW
    n_chunks = n_per_w // _CH
    n_trips = n_chunks // _NBUF
    assert n_chunks >= _NBUF + _A
    mesh = plsc.VectorSubcoreMesh(core_axis_name="c", subcore_axis_name="s")

    @functools.partial(
        pl.kernel,
        mesh=mesh,
        out_type=jax.ShapeDtypeStruct((B, D_MODEL), jnp.float32),
        scratch_types=[
            pltpu.VMEM((n_per_w,), jnp.int32),
            pltpu.VMEM((_NBUF, _CH, D_MODEL), jnp.float32),
            pltpu.SemaphoreType.DMA((_NBUF,)),
            pltpu.SemaphoreType.DMA((_NBUF,)),
        ],
    )
    def gather(x_hbm, table_hbm, out_hbm, idx_all, rows, gsem, osem):
        wid = lax.axis_index("s") * _NC + lax.axis_index("c")
        base = wid * n_per_w

        pltpu.sync_copy(x_hbm.at[pl.ds(base, n_per_w)], idx_all)

        def fire_gather(g, b):
            pltpu.async_copy(
                table_hbm.at[idx_all.at[pl.ds(g * _CH, _CH)]],
                rows.at[b], gsem.at[b])

        def wait_gather(g, b):
            pltpu.make_async_copy(
                table_hbm.at[idx_all.at[pl.ds(g * _CH, _CH)]],
                rows.at[b], gsem.at[b]).wait()

        def wait_out(b):
            pltpu.make_async_copy(
                rows.at[b], out_hbm.at[pl.ds(base, _CH)], osem.at[b]).wait()

        for b in range(_A):
            fire_gather(b, b)

        def trip_body(t, carry):
            for b in range(_NBUF):
                g = t * _NBUF + b
                off = base + g * _CH
                wait_gather(g, b)
                pltpu.async_copy(rows.at[b], out_hbm.at[pl.ds(off, _CH)],
                                 osem.at[b])

                bq = (b + _A) % _NBUF
                if b + _A < _NBUF:
                    @pl.when(t > 0)
                    def _wait_prev_out():
                        wait_out(bq)
                    fire_gather(g + _A, bq)
                else:
                    @pl.when(t < n_trips - 1)
                    def _prefetch_next_trip():
                        wait_out(bq)
                        fire_gather(g + _A, bq)
            return carry

        lax.fori_loop(0, n_trips, trip_body, 0, unroll=False)

        for k in range(_A):
            wait_out((n_chunks - _A + k) % _NBUF)

    return gather


def kernel(x, table):
    S, T = x.shape
    B = S * T
    V = table.shape[0]
    x_flat = x.reshape(B).astype(jnp.int32)
    scaled = _make_scale(V)(table)
    out = _make_gather(B)(x_flat, scaled)
    return out.reshape(S, T, D_MODEL)
